# baseline (speedup 1.0000x reference)
import jax
import jax.numpy as jnp
from jax import lax
from jax.experimental import pallas as pl
from jax.experimental.pallas import tpu as pltpu
from jax.experimental.pallas import tpu_sc as plsc

_NC = 2
_NS = 16
_NW = _NC * _NS
_LANES = 16

_B = 4096
_E = 64
_H = 50
_HP = 64
_CH = 2
_ROW = _CH * _HP
_B_PER_W = _B // _NW
_IDX_PER_W = _B_PER_W * _H
_CHUNKS = _B_PER_W // _CH
_NBUF = 4
_QS = _E // _LANES
_NPACK = _CHUNKS * _ROW // _LANES


def _gather_sum_body(idx_hbm, table_hbm, out_hbm,
                     raw_v, idx_v, rows_v, outb_v, s0, s1, s2, s3):
    sems = (s0, s1, s2, s3)
    wid = lax.axis_index("s") * _NC + lax.axis_index("c")
    base = pl.multiple_of(wid * _B_PER_W, 8)
    ibase = pl.multiple_of(wid * _IDX_PER_W, 8)

    pltpu.sync_copy(idx_hbm.at[pl.ds(ibase, _IDX_PER_W)], raw_v)

    lane = lax.iota(jnp.int32, _LANES)

    def pack(v, carry):
        p = v * _LANES + lane
        e = lax.shift_right_logical(p, 6)
        off = lax.bitwise_and(p, jnp.int32(_HP - 1))
        src = e * _H + off
        valid = off < _H
        src = jnp.minimum(src, jnp.int32(_IDX_PER_W - 1))
        g = plsc.load_gather(raw_v, [src])
        val = jnp.where(valid, g, jnp.int32(0))
        r = lax.shift_right_logical(v, 3)
        col = lax.bitwise_and(v, jnp.int32(7)) * _LANES
        idx_v[r, pl.ds(col, _LANES)] = val
        return carry

    plsc.parallel_loop(0, _NPACK, unroll=4, carry=jnp.int32(0))(pack)

    def gather_start(c, b):
        pltpu.async_copy(table_hbm.at[idx_v.at[c]], rows_v.at[b], sems[b])

    def gather_wait(c, b):
        pltpu.make_async_copy(
            table_hbm.at[idx_v.at[c]], rows_v.at[b], sems[b]
        ).wait()

    for b in range(_NBUF):
        gather_start(b, b)

    def reduce_elem(rb, e):
        zero = jnp.zeros((_LANES,), jnp.float32)
        init = (zero, zero, zero, zero)

        def red(j, acc):
            r = e * _HP + j
            return tuple(
                acc[q] + rb[r, pl.ds(q * _LANES, _LANES)] for q in range(_QS)
            )

        return plsc.parallel_loop(0, _H, unroll=10, carry=init)(red)

    def t_body(t, carry):
        for b in range(_NBUF):
            c = t * _NBUF + b
            gather_wait(c, b)
            for e in range(_CH):
                acc = reduce_elem(rows_v.at[b], e)
                row = c * _CH + e
                for q in range(_QS):
                    outb_v[row, pl.ds(q * _LANES, _LANES)] = acc[q]
            nc = c + _NBUF

            @pl.when(nc < _CHUNKS)
            def _():
                gather_start(nc, b)

        return carry

    lax.fori_loop(0, _CHUNKS // _NBUF, t_body, 0)

    pltpu.sync_copy(outb_v, out_hbm.at[pl.ds(base, _B_PER_W)])


def _gather_sum(idx_flat, table):
    k = pl.kernel(
        _gather_sum_body,
        out_type=jax.ShapeDtypeStruct((_B, _E), jnp.float32),
        mesh=plsc.VectorSubcoreMesh(
            core_axis_name="c", subcore_axis_name="s",
            num_cores=_NC, num_subcores=_NS,
        ),
        scratch_types=[
            pltpu.VMEM((_IDX_PER_W,), jnp.int32),
            pltpu.VMEM((_CHUNKS, _ROW), jnp.int32),
            pltpu.VMEM((_NBUF, _ROW, _E), jnp.float32),
            pltpu.VMEM((_B_PER_W, _E), jnp.float32),
            pltpu.SemaphoreType.DMA,
            pltpu.SemaphoreType.DMA,
            pltpu.SemaphoreType.DMA,
            pltpu.SemaphoreType.DMA,
        ],
        compiler_params=pltpu.CompilerParams(
            use_tc_tiling_on_sc=False, needs_layout_passes=False,
        ),
    )
    return k(idx_flat, table)


def _proj_body(x_ref, w_ref, b_ref, o_ref):
    x = jnp.maximum(x_ref[...], 0.0)
    o_ref[...] = (
        lax.dot_general(
            x, w_ref[...],
            dimension_numbers=(((1,), (1,)), ((), ())),
            preferred_element_type=jnp.float32,
        )
        + b_ref[...]
    )


def _proj(x, W, b2d):
    B, E = x.shape
    T = W.shape[0]
    blk = 512
    return pl.pallas_call(
        _proj_body,
        grid=(B // blk,),
        in_specs=[
            pl.BlockSpec((blk, E), lambda i: (i, 0)),
            pl.BlockSpec((T, E), lambda i: (0, 0)),
            pl.BlockSpec((1, T), lambda i: (0, 0)),
        ],
        out_specs=pl.BlockSpec((blk, T), lambda i: (i, 0)),
        out_shape=jax.ShapeDtypeStruct((B, T), jnp.float32),
    )(x, W, b2d)


def kernel(input_text, table, W, b):
    idx_flat = input_text.reshape(-1)
    sums = _gather_sum(idx_flat, table)
    return _proj(sums, W, b.reshape(1, -1))


# speedup vs baseline: 2.5867x; 2.5867x over previous
import jax
import jax.numpy as jnp
from jax import lax
from jax.experimental import pallas as pl
from jax.experimental.pallas import tpu as pltpu
from jax.experimental.pallas import tpu_sc as plsc

_NC = 2
_NS = 16
_NW = _NC * _NS
_LANES = 16

_B = 4096
_E = 64
_H = 50
_B_PER_W = _B // _NW
_CHUNKS = _B_PER_W
_NBUF = 4
_QS = _E // _LANES


def _gather_sum_body(idxf_hbm, table_hbm, out_hbm,
                     idxf_v, idx_v, rows_v, outb_v, s0, s1, s2, s3):
    sems = (s0, s1, s2, s3)
    wid = lax.axis_index("s") * _NC + lax.axis_index("c")
    base = pl.multiple_of(wid * _B_PER_W, 8)

    pltpu.sync_copy(idxf_hbm.at[pl.ds(base, _B_PER_W)], idxf_v)

    def conv(r, carry):
        for col in (0, 16, 32, _H - _LANES):
            idx_v[r, pl.ds(col, _LANES)] = (
                idxf_v[r, pl.ds(col, _LANES)].astype(jnp.int32)
            )
        return carry

    plsc.parallel_loop(0, _B_PER_W, unroll=4, carry=jnp.int32(0))(conv)

    def gather_start(c, b):
        pltpu.async_copy(table_hbm.at[idx_v.at[c]], rows_v.at[b], sems[b])

    def gather_wait(c, b):
        pltpu.make_async_copy(
            table_hbm.at[idx_v.at[c]], rows_v.at[b], sems[b]
        ).wait()

    for b in range(_NBUF):
        gather_start(b, b)

    def reduce_rows(rb):
        zero = jnp.zeros((_LANES,), jnp.float32)
        init = (zero, zero, zero, zero)

        def red(j, acc):
            return tuple(
                acc[q] + rb[j, pl.ds(q * _LANES, _LANES)] for q in range(_QS)
            )

        return plsc.parallel_loop(0, _H, unroll=10, carry=init)(red)

    def t_body(t, carry):
        for b in range(_NBUF):
            c = t * _NBUF + b
            gather_wait(c, b)
            acc = reduce_rows(rows_v.at[b])
            for q in range(_QS):
                outb_v[c, pl.ds(q * _LANES, _LANES)] = acc[q]
            nc = c + _NBUF

            @pl.when(nc < _CHUNKS)
            def _():
                gather_start(nc, b)

        return carry

    lax.fori_loop(0, _CHUNKS // _NBUF, t_body, 0)

    pltpu.sync_copy(outb_v, out_hbm.at[pl.ds(base, _B_PER_W)])


def _gather_sum(idx_f, table):
    k = pl.kernel(
        _gather_sum_body,
        out_type=jax.ShapeDtypeStruct((_B, _E), jnp.float32),
        mesh=plsc.VectorSubcoreMesh(
            core_axis_name="c", subcore_axis_name="s",
            num_cores=_NC, num_subcores=_NS,
        ),
        scratch_types=[
            pltpu.VMEM((_B_PER_W, _H), jnp.float32),
            pltpu.VMEM((_B_PER_W, _H), jnp.int32),
            pltpu.VMEM((_NBUF, _H, _E), jnp.float32),
            pltpu.VMEM((_B_PER_W, _E), jnp.float32),
            pltpu.SemaphoreType.DMA,
            pltpu.SemaphoreType.DMA,
            pltpu.SemaphoreType.DMA,
            pltpu.SemaphoreType.DMA,
        ],
        compiler_params=pltpu.CompilerParams(use_tc_tiling_on_sc=False),
    )
    return k(idx_f, table)


def _proj_body(x_ref, w_ref, b_ref, o_ref):
    x = jnp.maximum(x_ref[...], 0.0)
    o_ref[...] = (
        lax.dot_general(
            x, w_ref[...],
            dimension_numbers=(((1,), (1,)), ((), ())),
            preferred_element_type=jnp.float32,
        )
        + b_ref[...]
    )


def _proj(x, W, b2d):
    B, E = x.shape
    T = W.shape[0]
    blk = 512
    return pl.pallas_call(
        _proj_body,
        grid=(B // blk,),
        in_specs=[
            pl.BlockSpec((blk, E), lambda i: (i, 0)),
            pl.BlockSpec((T, E), lambda i: (0, 0)),
            pl.BlockSpec((1, T), lambda i: (0, 0)),
        ],
        out_specs=pl.BlockSpec((blk, T), lambda i: (i, 0)),
        out_shape=jax.ShapeDtypeStruct((B, T), jnp.float32),
    )(x, W, b2d)


def kernel(input_text, table, W, b):
    idx_f = input_text.astype(jnp.float32)
    sums = _gather_sum(idx_f, table)
    return _proj(sums, W, b.reshape(1, -1))


# speedup vs baseline: 2.9415x; 1.1372x over previous
import jax
import jax.numpy as jnp
from jax import lax
from jax.experimental import pallas as pl
from jax.experimental.pallas import tpu as pltpu
from jax.experimental.pallas import tpu_sc as plsc

_NC = 2
_NS = 16
_NW = _NC * _NS
_LANES = 16

_B = 4096
_E = 64
_H = 50
_B_PER_W = _B // _NW
_CHUNKS = _B_PER_W
_NBUF = 4
_QS = _E // _LANES


def _gather_sum_body(idx_hbm, table_hbm, out_hbm,
                     raw_v, idx_v, rows_v, outb_v, s0, s1, s2, s3):
    sems = (s0, s1, s2, s3)
    wid = lax.axis_index("s") * _NC + lax.axis_index("c")
    base = pl.multiple_of(wid * _B_PER_W, 8)

    pltpu.sync_copy(idx_hbm.at[pl.ds(base, _B_PER_W)], raw_v)

    def conv(r, carry):
        for col in (0, 16, 32, _H - _LANES):
            idx_v[r, pl.ds(col, _LANES)] = (
                raw_v[r, pl.ds(col, _LANES)] + raw_v[r, pl.ds(col, _LANES)]
            )
        return carry

    plsc.parallel_loop(0, _B_PER_W, unroll=4, carry=jnp.int32(0))(conv)

    def gather_start(c, b):
        pltpu.async_copy(table_hbm.at[idx_v.at[c]], rows_v.at[b], sems[b])

    def gather_wait(c, b):
        pltpu.make_async_copy(
            table_hbm.at[idx_v.at[c]], rows_v.at[b], sems[b]
        ).wait()

    for b in range(_NBUF):
        gather_start(b, b)

    def reduce_rows(rb):
        zero = jnp.zeros((_LANES,), jnp.float32)
        init = (zero, zero, zero, zero)

        def red(j, acc):
            return tuple(
                acc[q] + rb[j, pl.ds(q * _LANES, _LANES)] for q in range(_QS)
            )

        return plsc.parallel_loop(0, _H, unroll=10, carry=init)(red)

    def t_body(t, carry):
        for b in range(_NBUF):
            c = t * _NBUF + b
            gather_wait(c, b)
            acc = reduce_rows(rows_v.at[b])
            for q in range(_QS):
                outb_v[c, pl.ds(q * _LANES, _LANES)] = acc[q]
            nc = c + _NBUF

            @pl.when(nc < _CHUNKS)
            def _():
                gather_start(nc, b)

        return carry

    lax.fori_loop(0, _CHUNKS // _NBUF, t_body, 0)

    pltpu.sync_copy(outb_v, out_hbm.at[pl.ds(base, _B_PER_W)])


def _gather_sum(idx, table2):
    k = pl.kernel(
        _gather_sum_body,
        out_type=jax.ShapeDtypeStruct((_B, _E), jnp.float32),
        mesh=plsc.VectorSubcoreMesh(
            core_axis_name="c", subcore_axis_name="s",
            num_cores=_NC, num_subcores=_NS,
        ),
        scratch_types=[
            pltpu.VMEM((_B_PER_W, _H), jnp.int32),
            pltpu.VMEM((_B_PER_W, _H), jnp.int32),
            pltpu.VMEM((_NBUF, _H, _E), jnp.float32),
            pltpu.VMEM((_B_PER_W, _E), jnp.float32),
            pltpu.SemaphoreType.DMA,
            pltpu.SemaphoreType.DMA,
            pltpu.SemaphoreType.DMA,
            pltpu.SemaphoreType.DMA,
        ],
        compiler_params=pltpu.CompilerParams(use_tc_tiling_on_sc=False),
    )
    return k(idx, table2)


def _proj_body(x_ref, w_ref, b_ref, o_ref):
    x = jnp.maximum(x_ref[...], 0.0)
    o_ref[...] = (
        lax.dot_general(
            w_ref[...], x,
            dimension_numbers=(((1,), (1,)), ((), ())),
            preferred_element_type=jnp.float32,
        )
        + b_ref[...]
    )


def _proj_t(x, W, bcol):
    B, E = x.shape
    T = W.shape[0]
    blk = 512
    return pl.pallas_call(
        _proj_body,
        grid=(B // blk,),
        in_specs=[
            pl.BlockSpec((blk, E), lambda i: (i, 0)),
            pl.BlockSpec((T, E), lambda i: (0, 0)),
            pl.BlockSpec((T, 1), lambda i: (0, 0)),
        ],
        out_specs=pl.BlockSpec((T, blk), lambda i: (0, i)),
        out_shape=jax.ShapeDtypeStruct((T, B), jnp.float32),
    )(x, W, bcol)


def kernel(input_text, table, W, b):
    V, E = table.shape
    table2 = jnp.pad(table, ((0, 0), (0, E))).reshape(2 * V, E)
    sums = _gather_sum(input_text, table2)
    out_t = _proj_t(sums, W, b.reshape(-1, 1))
    return out_t.T
